# Initial kernel scaffold; baseline (speedup 1.0000x reference)
import jax
import jax.numpy as jnp
from jax.experimental import pallas as pl


def kernel(x, edge_index, edge_attr, edge_len, edge_W1, edge_W2, edge_gamma, edge_beta, node_W1, node_W2, node_gamma, node_beta):
    raise NotImplementedError("write your pallas kernel here")


# speedup vs baseline: 2.5527x; 2.5527x over previous
import functools

import jax
import jax.numpy as jnp
from jax import lax
from jax.experimental import pallas as pl
from jax.experimental.pallas import tpu as pltpu
from jax.experimental.pallas import tpu_sc as plsc

_NC = 2
_NS = 16
_NW = _NC * _NS
_L = 16

_CH = 80


def _proj_body(x_ref, w1a_ref, w1b_ref, wn_ref, a_ref, b_ref, c_ref):
    xv = x_ref[...]
    a_ref[...] = jnp.dot(xv, w1a_ref[...], preferred_element_type=jnp.float32)
    b_ref[...] = jnp.dot(xv, w1b_ref[...], preferred_element_type=jnp.float32)
    c_ref[...] = jnp.dot(xv, wn_ref[...], preferred_element_type=jnp.float32)


def _eprep_body(ea_ref, w2k_ref, q_ref):
    q_ref[...] = jnp.dot(ea_ref[...], w2k_ref[...],
                         preferred_element_type=jnp.float32)


def _edge_sc_body(epw, nchunk,
                  ni_hbm, nj_hbm, a_hbm, b_hbm, q_hbm, el_hbm, w1c_hbm,
                  er_hbm, st_hbm,
                  idx_i, idx_j, av, bv, qv, elv, ev, w1cv, stv, sema, semb):
    wid = lax.axis_index("s") * _NC + lax.axis_index("c")
    base = wid * epw
    pltpu.sync_copy(w1c_hbm, w1cv)
    w1c = w1cv[0]

    def chunk(ci, carry):
        sm, ss = carry
        cb = base + ci * _CH
        pltpu.sync_copy(ni_hbm.at[pl.ds(cb, _CH)], idx_i)
        pltpu.sync_copy(nj_hbm.at[pl.ds(cb, _CH)], idx_j)
        ga = pltpu.async_copy(a_hbm.at[idx_i], av, sema)
        gb = pltpu.async_copy(b_hbm.at[idx_j], bv, semb)
        pltpu.sync_copy(q_hbm.at[pl.ds(cb, _CH)], qv)
        pltpu.sync_copy(el_hbm.at[pl.ds(cb, _CH)], elv)
        ga.wait()
        gb.wait()

        def gbody(g, cc):
            sm2, ss2 = cc
            eb = g * _L
            lv = elv[pl.ds(eb, _L)]
            for k in range(_L):
                e = eb + k
                z = (av[e] + bv[e] + lv[k] * w1c) * qv[e]
                er = z / (1.0 + jnp.exp(-z))
                ev[e] = er
                sm2 = sm2 + er
                ss2 = ss2 + er * er
            return (sm2, ss2)

        sm, ss = lax.fori_loop(0, _CH // _L, gbody, (sm, ss))
        pltpu.sync_copy(ev, er_hbm.at[pl.ds(cb, _CH)])
        return (sm, ss)

    z16 = jnp.zeros((_L,), jnp.float32)
    sm, ss = lax.fori_loop(0, nchunk, chunk, (z16, z16))
    for k in range(8):
        stv[0, pl.ds(k * _L, _L)] = sm
        stv[1, pl.ds(k * _L, _L)] = ss
    pltpu.sync_copy(stv, st_hbm.at[wid])


def _fin_body(e_total, er_ref, ea_ref, sp_ref, ssp_ref, g_ref, be_ref,
              wk_ref, e_ref, t_ref):
    sm = jnp.sum(sp_ref[...], axis=0, keepdims=True)
    ss = jnp.sum(ssp_ref[...], axis=0, keepdims=True)
    mu = sm / e_total
    var = ss / e_total - mu * mu
    rstd = lax.rsqrt(var + 1e-5)
    ev = (er_ref[...] - mu) * (rstd * g_ref[...]) + be_ref[...] + ea_ref[...]
    e_ref[...] = ev
    t_ref[...] = jnp.dot(ev, wk_ref[...], preferred_element_type=jnp.float32)


def _agg_sc_body(epw, nchunk, rows_per_tile,
                 ni_hbm, c_hbm, t_hbm, z_hbm,
                 aggp_hbm,
                 idx, cv, tv, mv, shared, semc):
    cid = lax.axis_index("c")
    sid = lax.axis_index("s")
    wid = sid * _NC + cid
    base = wid * epw
    d_node = c_hbm.shape[1]
    nh = d_node // _L

    zr = z_hbm.shape[0]
    tbase = sid * rows_per_tile
    for k in range(rows_per_tile // zr):
        pltpu.sync_copy(z_hbm, shared.at[pl.ds(tbase + k * zr, zr)])
    plsc.subcore_barrier()

    def chunk(ci, carry):
        cb = base + ci * _CH
        pltpu.sync_copy(ni_hbm.at[pl.ds(cb, _CH)], idx)
        gc = pltpu.async_copy(c_hbm.at[idx], cv, semc)
        pltpu.sync_copy(t_hbm.at[pl.ds(cb, _CH)], tv)
        gc.wait()

        def ebody(e, cc):
            for h in range(nh):
                z = cv[e, pl.ds(h * _L, _L)] * tv[e, pl.ds(h * _L, _L)]
                mv[e, pl.ds(h * _L, _L)] = z / (1.0 + jnp.exp(-z))
            return cc

        lax.fori_loop(0, _CH, ebody, 0)
        pltpu.sync_copy(mv, shared.at[idx], add=True)
        return carry

    lax.fori_loop(0, nchunk, chunk, 0)
    plsc.subcore_barrier()
    pltpu.sync_copy(shared.at[pl.ds(tbase, rows_per_tile)],
                    aggp_hbm.at[cid, pl.ds(tbase, rows_per_tile)])


def _node_body(p_ref, x_ref, g_ref, be_ref, n_ref):
    agg = p_ref[0] + p_ref[1]
    mu = jnp.mean(agg, axis=0, keepdims=True)
    var = jnp.mean((agg - mu) * (agg - mu), axis=0, keepdims=True)
    n_ref[...] = ((agg - mu) * lax.rsqrt(var + 1e-5) * g_ref[...]
                  + be_ref[...] + x_ref[...])


def kernel(x, edge_index, edge_attr, edge_len, edge_W1, edge_W2, edge_gamma,
           edge_beta, node_W1, node_W2, node_gamma, node_beta):
    n_nodes, d_node = x.shape
    n_edges, d_edge = edge_attr.shape
    epw = n_edges // _NW
    nchunk = epw // _CH
    rows_per_tile = n_nodes // _NS
    pack = 128 // d_edge
    e8 = n_edges // pack

    ni = edge_index[0]
    nj = edge_index[1]
    w1a = edge_W1[:d_node]
    w1b = edge_W1[d_node:2 * d_node]
    w1c = edge_W1[2 * d_node:]
    eye = jnp.eye(pack, dtype=jnp.float32)
    w2k = jnp.kron(eye, edge_W2)
    wnk = jnp.kron(eye, node_W2)
    ea8 = edge_attr.reshape(e8, pack * d_edge)

    a_t, b_t, c_t = pl.pallas_call(
        _proj_body,
        out_shape=(
            jax.ShapeDtypeStruct((n_nodes, d_edge), jnp.float32),
            jax.ShapeDtypeStruct((n_nodes, d_edge), jnp.float32),
            jax.ShapeDtypeStruct((n_nodes, d_node), jnp.float32),
        ),
    )(x, w1a, w1b, node_W1)

    be1 = 8000
    grid1 = e8 // be1
    q_t = pl.pallas_call(
        _eprep_body,
        grid=(grid1,),
        in_specs=[
            pl.BlockSpec((be1, 128), lambda i: (i, 0)),
            pl.BlockSpec((128, 128), lambda i: (0, 0)),
        ],
        out_specs=pl.BlockSpec((be1, 128), lambda i: (i, 0)),
        out_shape=jax.ShapeDtypeStruct((e8, 128), jnp.float32),
    )(ea8, w2k)

    mesh = plsc.VectorSubcoreMesh(core_axis_name="c", subcore_axis_name="s")
    sc_params = pltpu.CompilerParams(use_tc_tiling_on_sc=False)
    er_t, st_t = pl.kernel(
        functools.partial(_edge_sc_body, epw, nchunk),
        compiler_params=sc_params,
        out_type=(
            jax.ShapeDtypeStruct((n_edges, d_edge), jnp.float32),
            jax.ShapeDtypeStruct((_NW, 2, 128), jnp.float32),
        ),
        mesh=mesh,
        scratch_types=[
            pltpu.VMEM((_CH,), jnp.int32),
            pltpu.VMEM((_CH,), jnp.int32),
            pltpu.VMEM((_CH, d_edge), jnp.float32),
            pltpu.VMEM((_CH, d_edge), jnp.float32),
            pltpu.VMEM((_CH, d_edge), jnp.float32),
            pltpu.VMEM((_CH,), jnp.float32),
            pltpu.VMEM((_CH, d_edge), jnp.float32),
            pltpu.VMEM((1, d_edge), jnp.float32),
            pltpu.VMEM((2, 128), jnp.float32),
            pltpu.SemaphoreType.DMA,
            pltpu.SemaphoreType.DMA,
        ],
    )(ni, nj, a_t, b_t, q_t.reshape(n_edges, d_edge), edge_len.reshape(-1),
      w1c)

    be3 = 2000
    grid3 = e8 // be3
    g128 = jnp.tile(edge_gamma, pack).reshape(1, 128)
    bt128 = jnp.tile(edge_beta, pack).reshape(1, 128)
    e_t, t_t = pl.pallas_call(
        functools.partial(_fin_body, float(n_edges)),
        grid=(grid3,),
        in_specs=[
            pl.BlockSpec((be3, 128), lambda i: (i, 0)),
            pl.BlockSpec((be3, 128), lambda i: (i, 0)),
            pl.BlockSpec((_NW, 128), lambda i: (0, 0)),
            pl.BlockSpec((_NW, 128), lambda i: (0, 0)),
            pl.BlockSpec((1, 128), lambda i: (0, 0)),
            pl.BlockSpec((1, 128), lambda i: (0, 0)),
            pl.BlockSpec((128, pack * d_node), lambda i: (0, 0)),
        ],
        out_specs=(
            pl.BlockSpec((be3, 128), lambda i: (i, 0)),
            pl.BlockSpec((be3, pack * d_node), lambda i: (i, 0)),
        ),
        out_shape=(
            jax.ShapeDtypeStruct((e8, 128), jnp.float32),
            jax.ShapeDtypeStruct((e8, pack * d_node), jnp.float32),
        ),
    )(er_t.reshape(e8, 128), ea8, st_t[:, 0, :], st_t[:, 1, :],
      g128, bt128, wnk)

    zrows = 125
    zeros_init = jnp.zeros((zrows, d_node), jnp.float32)
    aggp = pl.kernel(
        functools.partial(_agg_sc_body, epw, nchunk, rows_per_tile),
        out_type=jax.ShapeDtypeStruct((_NC, n_nodes, d_node), jnp.float32),
        compiler_params=sc_params,
        mesh=mesh,
        scratch_types=[
            pltpu.VMEM((_CH,), jnp.int32),
            pltpu.VMEM((_CH, d_node), jnp.float32),
            pltpu.VMEM((_CH, d_node), jnp.float32),
            pltpu.VMEM((_CH, d_node), jnp.float32),
            pltpu.VMEM_SHARED((n_nodes, d_node), jnp.float32),
            pltpu.SemaphoreType.DMA,
        ],
    )(ni, c_t, t_t.reshape(n_edges, d_node), zeros_init)

    n_t = pl.pallas_call(
        _node_body,
        out_shape=jax.ShapeDtypeStruct((n_nodes, d_node), jnp.float32),
    )(aggp, x, node_gamma.reshape(1, -1), node_beta.reshape(1, -1))

    return (n_t, e_t.reshape(n_edges, d_edge))
